# baseline (speedup 1.0000x reference)
import jax
import jax.numpy as jnp
from jax import lax
from jax.experimental import pallas as pl
from jax.experimental.pallas import tpu as pltpu
from jax.experimental.pallas import tpu_sc as plsc

BATCH = 16384
K = 128
_INFO = plsc.get_sparse_core_info()
_NC = _INFO.num_cores
_NW = _NC * _INFO.num_subcores
ROWS = BATCH // _NW


def _sc_body(x_hbm, w_hbm, b_hbm, out_hbm, x_v, w_v, b_v, out_v):
    wid = lax.axis_index("s") * _NC + lax.axis_index("c")
    base = wid * ROWS
    b_v[...] = jnp.zeros((16,), jnp.float32)
    pltpu.sync_copy(w_hbm, w_v)
    pltpu.sync_copy(b_hbm, b_v.at[pl.ds(0, 1)])
    pltpu.sync_copy(x_hbm.at[pl.ds(base, ROWS)], x_v)

    wchunks = [w_v[0, pl.ds(16 * k, 16)] for k in range(K // 16)]
    bias_vec = b_v[...]
    lane = lax.iota(jnp.int32, 16)
    last_lane = lane == 15

    def row(r, carry):
        s = bias_vec + x_v[r, pl.ds(0, 16)] * wchunks[0]
        for k in range(1, K // 16):
            s = s + x_v[r, pl.ds(16 * k, 16)] * wchunks[k]
        total = plsc.cumsum(s)
        plsc.store_scatter(
            out_v, [jnp.full((16,), r, jnp.int32)], total, mask=last_lane)
        return carry

    lax.fori_loop(0, ROWS, row, 0, unroll=4)
    pltpu.sync_copy(out_v, out_hbm.at[pl.ds(base, ROWS)])


def kernel(x_cont, W, b):
    mesh = plsc.VectorSubcoreMesh(core_axis_name="c", subcore_axis_name="s")
    f = pl.kernel(
        _sc_body,
        mesh=mesh,
        compiler_params=pltpu.CompilerParams(needs_layout_passes=False),
        out_type=jax.ShapeDtypeStruct((BATCH,), jnp.float32),
        scratch_types=[
            pltpu.VMEM((ROWS, K), jnp.float32),
            pltpu.VMEM((1, K), jnp.float32),
            pltpu.VMEM((16,), jnp.float32),
            pltpu.VMEM((ROWS,), jnp.float32),
        ],
    )
    return f(x_cont, W, b).reshape(BATCH, 1)


# speedup vs baseline: 1.2613x; 1.2613x over previous
import jax
import jax.numpy as jnp
from jax import lax
from jax.experimental import pallas as pl
from jax.experimental.pallas import tpu as pltpu
from jax.experimental.pallas import tpu_sc as plsc

BATCH = 16384
K = 128
_INFO = plsc.get_sparse_core_info()
_NC = _INFO.num_cores
_NW = _NC * _INFO.num_subcores
ROWS = BATCH // _NW


def _dyn_gather(v, idx):
    return lax.gather(
        v, idx[:, None],
        lax.GatherDimensionNumbers(
            offset_dims=(), collapsed_slice_dims=(0,), start_index_map=(0,)),
        (1,), mode=lax.GatherScatterMode.PROMISE_IN_BOUNDS)


def _sc_body(x_hbm, w_hbm, b_hbm, out_hbm, x_v, w_v, b_v, out_v):
    wid = lax.axis_index("s") * _NC + lax.axis_index("c")
    base = wid * ROWS
    pltpu.sync_copy(w_hbm, w_v)
    pltpu.sync_copy(b_hbm, b_v.at[pl.ds(0, 1)])
    pltpu.sync_copy(x_hbm.at[pl.ds(base, ROWS)], x_v)

    wchunks = [w_v[0, pl.ds(16 * k, 16)] for k in range(K // 16)]
    lane = lax.iota(jnp.int32, 16)
    zeros_i = jnp.zeros((16,), jnp.int32)
    bias_splat = plsc.load_gather(b_v, [zeros_i])
    onehot = [(lane == r).astype(jnp.float32) for r in range(16)]
    perms = [lax.iota(jnp.int32, 16) ^ d for d in (1, 2, 4, 8)]

    def _tree_sum(vs):
        while len(vs) > 1:
            vs = [a + b for a, b in zip(vs[::2], vs[1::2])]
        return vs[0]

    def _hsum_splat(s):
        for p in perms:
            s = s + _dyn_gather(s, p)
        return s

    @plsc.parallel_loop(0, ROWS // 16, carry=jnp.int32(0))
    def group(g, carry):
        rbase = g * 16
        parts = []
        for r in range(16):
            prods = [x_v[rbase + r, pl.ds(16 * k, 16)] * wchunks[k]
                     for k in range(K // 16)]
            parts.append(_hsum_splat(_tree_sum(prods)) * onehot[r])
        out_v[pl.ds(rbase, 16)] = bias_splat + _tree_sum(parts)
        return carry

    pltpu.sync_copy(out_v, out_hbm.at[pl.ds(base, ROWS)])


def kernel(x_cont, W, b):
    mesh = plsc.VectorSubcoreMesh(core_axis_name="c", subcore_axis_name="s")
    f = pl.kernel(
        _sc_body,
        mesh=mesh,
        compiler_params=pltpu.CompilerParams(needs_layout_passes=False),
        out_type=jax.ShapeDtypeStruct((BATCH,), jnp.float32),
        scratch_types=[
            pltpu.VMEM((ROWS, K), jnp.float32),
            pltpu.VMEM((1, K), jnp.float32),
            pltpu.VMEM((16,), jnp.float32),
            pltpu.VMEM((ROWS,), jnp.float32),
        ],
    )
    return f(x_cont, W, b).reshape(BATCH, 1)
